# baseline (speedup 1.0000x reference)
import functools

import jax
import jax.numpy as jnp
from jax import lax
from jax.experimental import pallas as pl
from jax.experimental.pallas import tpu as pltpu
from jax.experimental.pallas import tpu_sc as plsc

VOCAB = 28996
DIM = 768
B_TOTAL = 4096 * 20
NUM_WORKERS = 32
PER_W = B_TOTAL // NUM_WORKERS
CHUNK = 128
N_CHUNKS = PER_W // CHUNK

_mesh = plsc.VectorSubcoreMesh(core_axis_name="c", subcore_axis_name="s")


@functools.partial(
    pl.kernel,
    mesh=_mesh,
    out_type=jax.ShapeDtypeStruct((B_TOTAL, DIM), jnp.float32),
    scratch_types=[
        pltpu.VMEM((PER_W,), jnp.int32),
        pltpu.VMEM((CHUNK, DIM), jnp.float32),
        pltpu.SemaphoreType.DMA,
    ],
)
def _gather_kernel(ids_hbm, table_hbm, out_hbm, idx_v, rows_v, sem):
    wid = lax.axis_index("s") * 2 + lax.axis_index("c")
    base = wid * PER_W
    pltpu.sync_copy(ids_hbm.at[pl.ds(base, PER_W)], idx_v)

    def body(g, carry):
        off = g * CHUNK
        pltpu.async_copy(
            table_hbm.at[idx_v.at[pl.ds(off, CHUNK)]], rows_v, sem
        )
        return carry

    lax.fori_loop(0, N_CHUNKS, body, 0)

    def drain(g, carry):
        off = g * CHUNK
        pltpu.make_async_copy(
            table_hbm.at[idx_v.at[pl.ds(off, CHUNK)]], rows_v, sem
        ).wait()
        return carry

    lax.fori_loop(0, N_CHUNKS, drain, 0)
    pltpu.sync_copy(rows_v, out_hbm.at[pl.ds(base, CHUNK)])


def kernel(input_ids, embeddings):
    b, t = input_ids.shape
    ids = input_ids.T.reshape(-1).astype(jnp.int32)
    out = _gather_kernel(ids, embeddings)
    return out.reshape(t, b, DIM).transpose(1, 0, 2)
